# baseline (speedup 1.0000x reference)
import functools
import jax
import jax.numpy as jnp
from jax import lax
from jax.experimental import pallas as pl
from jax.experimental.pallas import tpu as pltpu
from jax.experimental.pallas import tpu_sc as plsc

NC, NS = 2, 16
NW = NC * NS
D = 64
CHUNK = 128
NBUF = 8
LAG = 4
TBLK = 1024


def _transpose_table(table_t):
    n = table_t.shape[1]
    grid = (n + TBLK - 1) // TBLK

    def body(in_ref, out_ref):
        x = in_ref[...]
        out_ref[...] = pltpu.einshape("a(bc)->b(ca)", x, b=TBLK // 2, c=2)

    return pl.pallas_call(
        body,
        grid=(grid,),
        in_specs=[pl.BlockSpec((D, TBLK), lambda g: (0, g))],
        out_specs=pl.BlockSpec((TBLK // 2, 2 * D), lambda g: (g, 0)),
        out_shape=jax.ShapeDtypeStruct((n // 2, 2 * D), jnp.float32),
    )(table_t)


@jax.jit
def _gather_rows(idx_t, table):
    S, B = idx_t.shape
    b_per_w = B // NW
    groups = b_per_w // CHUNK
    n_chunks = S * groups

    mesh = plsc.VectorSubcoreMesh(
        core_axis_name="c", subcore_axis_name="s",
        num_cores=NC, num_subcores=NS)

    @functools.partial(
        pl.kernel,
        out_type=jax.ShapeDtypeStruct((S, B, D), jnp.float32),
        mesh=mesh,
        scratch_types=[
            pltpu.VMEM((S, b_per_w), jnp.int32),
            pltpu.VMEM((NBUF, CHUNK, D), jnp.float32),
        ] + [pltpu.SemaphoreType.DMA] * (2 * NBUF),
        compiler_params=pltpu.CompilerParams(use_tc_tiling_on_sc=False),
    )
    def k(idx_hbm, table_hbm, out_hbm, idx_v, rows_v, *sems):
        gsems = sems[:NBUF]
        ssems = sems[NBUF:]
        wid = lax.axis_index("s") * NC + lax.axis_index("c")
        b_base = wid * b_per_w
        pltpu.sync_copy(idx_hbm.at[:, pl.ds(b_base, b_per_w)], idx_v)

        def chunk_pos(q):
            return q // groups, (q % groups) * CHUNK

        def start_gather(q, b):
            s, boff = chunk_pos(q)
            pltpu.async_copy(
                table_hbm.at[idx_v.at[s, pl.ds(boff, CHUNK)]],
                rows_v.at[b], gsems[b])

        def wait_gather(b):
            pltpu.make_async_copy(
                table_hbm.at[idx_v.at[0, pl.ds(0, CHUNK)]],
                rows_v.at[b], gsems[b]).wait()

        def out_slice(q):
            s, boff = chunk_pos(q)
            return out_hbm.at[s, pl.ds(b_base + boff, CHUNK)]

        def start_store(q, b):
            pltpu.async_copy(rows_v.at[b], out_slice(q), ssems[b])

        def wait_store(q, b):
            pltpu.make_async_copy(rows_v.at[b], out_slice(q), ssems[b]).wait()

        for b in range(LAG):
            start_gather(b, b)
        for q in range(LAG):
            start_gather(q + LAG, q + LAG)
            wait_gather(q)
            start_store(q, q)

        @pl.loop(LAG, n_chunks - LAG, step=NBUF)
        def body(g):
            for i in range(NBUF):
                q = g + i
                mq = (LAG + i) % NBUF
                mg = (2 * LAG + i) % NBUF
                wait_store(q - (NBUF - LAG), mg)
                start_gather(q + LAG, mg)
                wait_gather(mq)
                start_store(q, mq)

        for j in range(n_chunks - LAG, n_chunks):
            b = j % NBUF
            wait_gather(b)
            start_store(j, b)
        for j in range(n_chunks - NBUF, n_chunks):
            wait_store(j, j % NBUF)

    return k(idx_t, table)


@jax.jit
def _embed(token_ids, embeddings):
    idx_t = token_ids.astype(jnp.int32).T
    table_pairs = _transpose_table(embeddings.T)
    table = table_pairs.reshape(-1, D)
    out = _gather_rows(idx_t, table)
    return out.transpose(1, 0, 2)


def kernel(token_ids, embeddings):
    return _embed(token_ids, embeddings)


# speedup vs baseline: 7.7585x; 7.7585x over previous
import functools
import jax
import jax.numpy as jnp
from jax import lax
from jax.experimental import pallas as pl
from jax.experimental.pallas import tpu as pltpu
from jax.experimental.pallas import tpu_sc as plsc

NC, NS = 2, 16
NW = NC * NS
D = 64
CHUNK = 128
NBUF = 8
LAG = 4
@functools.partial(jax.jit, static_argnames=("s_lo", "s_len"))
def _gather_rows(idx_t, table, s_lo, s_len):
    S, B = idx_t.shape
    b_per_w = B // NW
    groups = b_per_w // CHUNK
    n_chunks = s_len * groups

    mesh = plsc.VectorSubcoreMesh(
        core_axis_name="c", subcore_axis_name="s",
        num_cores=NC, num_subcores=NS)

    @functools.partial(
        pl.kernel,
        out_type=jax.ShapeDtypeStruct((s_len, B, D), jnp.float32),
        mesh=mesh,
        scratch_types=[
            pltpu.VMEM((s_len, b_per_w), jnp.int32),
            pltpu.VMEM((NBUF, CHUNK, D), jnp.float32),
        ] + [pltpu.SemaphoreType.DMA] * (2 * NBUF),
        compiler_params=pltpu.CompilerParams(use_tc_tiling_on_sc=False),
    )
    def k(idx_hbm, table_hbm, out_hbm, idx_v, rows_v, *sems):
        gsems = sems[:NBUF]
        ssems = sems[NBUF:]
        wid = lax.axis_index("s") * NC + lax.axis_index("c")
        b_base = wid * b_per_w
        pltpu.sync_copy(
            idx_hbm.at[pl.ds(s_lo, s_len), pl.ds(b_base, b_per_w)], idx_v)

        def chunk_pos(q):
            return q // groups, (q % groups) * CHUNK

        def start_gather(q, b):
            s, boff = chunk_pos(q)
            pltpu.async_copy(
                table_hbm.at[idx_v.at[s, pl.ds(boff, CHUNK)]],
                rows_v.at[b], gsems[b])

        def wait_gather(b):
            pltpu.make_async_copy(
                table_hbm.at[idx_v.at[0, pl.ds(0, CHUNK)]],
                rows_v.at[b], gsems[b]).wait()

        def out_slice(q):
            s, boff = chunk_pos(q)
            return out_hbm.at[s, pl.ds(b_base + boff, CHUNK)]

        def start_store(q, b):
            pltpu.async_copy(rows_v.at[b], out_slice(q), ssems[b])

        def wait_store(q, b):
            pltpu.make_async_copy(rows_v.at[b], out_slice(q), ssems[b]).wait()

        for b in range(LAG):
            start_gather(b, b)
        for q in range(LAG):
            start_gather(q + LAG, q + LAG)
            wait_gather(q)
            start_store(q, q)

        main = ((n_chunks - 2 * LAG) // NBUF) * NBUF

        @pl.loop(LAG, LAG + main, step=NBUF)
        def body(g):
            for i in range(NBUF):
                q = g + i
                mq = (LAG + i) % NBUF
                mg = (2 * LAG + i) % NBUF
                wait_store(q - (NBUF - LAG), mg)
                start_gather(q + LAG, mg)
                wait_gather(mq)
                start_store(q, mq)

        for q in range(LAG + main, n_chunks - LAG):
            mq = q % NBUF
            mg = (q + LAG) % NBUF
            wait_store(q - (NBUF - LAG), mg)
            start_gather(q + LAG, mg)
            wait_gather(mq)
            start_store(q, mq)

        for j in range(n_chunks - LAG, n_chunks):
            b = j % NBUF
            wait_gather(b)
            start_store(j, b)
        for j in range(n_chunks - NBUF, n_chunks):
            wait_store(j, j % NBUF)

    return k(idx_t, table)


NSPLIT = 1


@jax.jit
def _embed(token_ids, embeddings):
    idx_t = token_ids.astype(jnp.int32).T
    S = idx_t.shape[0]
    bounds = [S * i // NSPLIT for i in range(NSPLIT + 1)]
    parts = [
        _gather_rows(idx_t, embeddings, s_lo=lo, s_len=hi - lo)
        for lo, hi in zip(bounds[:-1], bounds[1:])
    ]
    out = jnp.concatenate(parts, axis=0) if len(parts) > 1 else parts[0]
    return out.transpose(1, 0, 2)


def kernel(token_ids, embeddings):
    return _embed(token_ids, embeddings)


# speedup vs baseline: 7.8205x; 1.0080x over previous
import functools
import jax
import jax.numpy as jnp
from jax import lax
from jax.experimental import pallas as pl
from jax.experimental.pallas import tpu as pltpu
from jax.experimental.pallas import tpu_sc as plsc

NC, NS = 2, 16
NW = NC * NS
D = 64
CHUNK = 128
NBUF = 8
LAG = 4
@functools.partial(jax.jit, static_argnames=("s_lo", "s_len"))
def _gather_rows(idx_t, table, s_lo, s_len):
    S, B = idx_t.shape
    b_per_w = B // NW
    groups = b_per_w // CHUNK
    n_chunks = s_len * groups

    mesh = plsc.VectorSubcoreMesh(
        core_axis_name="c", subcore_axis_name="s",
        num_cores=NC, num_subcores=NS)

    @functools.partial(
        pl.kernel,
        out_type=jax.ShapeDtypeStruct((s_len, B, D), jnp.float32),
        mesh=mesh,
        scratch_types=[
            pltpu.VMEM((s_len, b_per_w), jnp.int32),
            pltpu.VMEM((NBUF, CHUNK, D), jnp.float32),
        ] + [pltpu.SemaphoreType.DMA] * (2 * NBUF),
        compiler_params=pltpu.CompilerParams(use_tc_tiling_on_sc=False),
    )
    def k(idx_hbm, table_hbm, out_hbm, idx_v, rows_v, *sems):
        gsems = sems[:NBUF]
        ssems = sems[NBUF:]
        wid = lax.axis_index("s") * NC + lax.axis_index("c")
        b_base = wid * b_per_w
        pltpu.sync_copy(
            idx_hbm.at[pl.ds(s_lo, s_len), pl.ds(b_base, b_per_w)], idx_v)

        def chunk_pos(q):
            return q // groups, (q % groups) * CHUNK

        def start_gather(q, b):
            s, boff = chunk_pos(q)
            pltpu.async_copy(
                table_hbm.at[idx_v.at[s, pl.ds(boff, CHUNK)]],
                rows_v.at[b], gsems[b])

        def wait_gather(q, b):
            s, boff = chunk_pos(q)
            pltpu.make_async_copy(
                table_hbm.at[idx_v.at[s, pl.ds(boff, CHUNK)]],
                rows_v.at[b], gsems[b]).wait()

        def out_slice(q):
            s, boff = chunk_pos(q)
            return out_hbm.at[s, pl.ds(b_base + boff, CHUNK)]

        def start_store(q, b):
            pltpu.async_copy(rows_v.at[b], out_slice(q), ssems[b])

        def wait_store(q, b):
            pltpu.make_async_copy(rows_v.at[b], out_slice(q), ssems[b]).wait()

        for b in range(LAG):
            start_gather(b, b)
        for q in range(LAG):
            start_gather(q + LAG, q + LAG)
            wait_gather(q, q)
            start_store(q, q)

        main = ((n_chunks - 2 * LAG) // NBUF) * NBUF

        @pl.loop(LAG, LAG + main, step=NBUF)
        def body(g):
            for i in range(NBUF):
                q = g + i
                mq = (LAG + i) % NBUF
                mg = (2 * LAG + i) % NBUF
                wait_store(q - (NBUF - LAG), mg)
                start_gather(q + LAG, mg)
                wait_gather(q, mq)
                start_store(q, mq)

        for q in range(LAG + main, n_chunks - LAG):
            mq = q % NBUF
            mg = (q + LAG) % NBUF
            wait_store(q - (NBUF - LAG), mg)
            start_gather(q + LAG, mg)
            wait_gather(q, mq)
            start_store(q, mq)

        for j in range(n_chunks - LAG, n_chunks):
            b = j % NBUF
            wait_gather(j, b)
            start_store(j, b)
        for j in range(n_chunks - NBUF, n_chunks):
            wait_store(j, j % NBUF)

    return k(idx_t, table)


NSPLIT = 1


@jax.jit
def _embed(token_ids, embeddings):
    idx_t = token_ids.astype(jnp.int32).T
    S = idx_t.shape[0]
    bounds = [S * i // NSPLIT for i in range(NSPLIT + 1)]
    parts = [
        _gather_rows(idx_t, embeddings, s_lo=lo, s_len=hi - lo)
        for lo, hi in zip(bounds[:-1], bounds[1:])
    ]
    out = jnp.concatenate(parts, axis=0) if len(parts) > 1 else parts[0]
    return out.transpose(1, 0, 2)


def kernel(token_ids, embeddings):
    return _embed(token_ids, embeddings)
